# baseline (speedup 1.0000x reference)
import functools
import math

import jax
import jax.numpy as jnp
from jax import lax
from jax.experimental import pallas as pl
from jax.experimental.pallas import tpu as pltpu
from jax.experimental.pallas import tpu_sc as plsc

D_MODEL = 64
SCALE = math.sqrt(D_MODEL)
CHUNK = 128
NBUF = 8


@functools.cache
def _build_gather(B, V):
    info = plsc.get_sparse_core_info()
    nc, ns = info.num_cores, info.num_subcores
    nw = nc * ns
    b_per_w = B // nw
    n_chunks = b_per_w // CHUNK
    assert n_chunks % NBUF == 0 and n_chunks >= 2 * NBUF
    mesh = plsc.VectorSubcoreMesh(core_axis_name="c", subcore_axis_name="s")

    @functools.partial(
        pl.kernel,
        mesh=mesh,
        compiler_params=pltpu.CompilerParams(use_tc_tiling_on_sc=False),
        out_type=jax.ShapeDtypeStruct((B, D_MODEL), jnp.float32),
        scratch_types=[
            pltpu.VMEM((n_chunks, CHUNK), jnp.int32),
            pltpu.VMEM((CHUNK, D_MODEL), jnp.float32),
            pltpu.VMEM((CHUNK, D_MODEL), jnp.float32),
            pltpu.VMEM((CHUNK, D_MODEL), jnp.float32),
            pltpu.VMEM((CHUNK, D_MODEL), jnp.float32),
            pltpu.VMEM((CHUNK, D_MODEL), jnp.float32),
            pltpu.VMEM((CHUNK, D_MODEL), jnp.float32),
            pltpu.VMEM((CHUNK, D_MODEL), jnp.float32),
            pltpu.VMEM((CHUNK, D_MODEL), jnp.float32),
            pltpu.SemaphoreType.DMA,
            pltpu.SemaphoreType.DMA,
            pltpu.SemaphoreType.DMA,
            pltpu.SemaphoreType.DMA,
            pltpu.SemaphoreType.DMA,
            pltpu.SemaphoreType.DMA,
            pltpu.SemaphoreType.DMA,
            pltpu.SemaphoreType.DMA,
            pltpu.SemaphoreType.DMA,
            pltpu.SemaphoreType.DMA,
            pltpu.SemaphoreType.DMA,
            pltpu.SemaphoreType.DMA,
            pltpu.SemaphoreType.DMA,
            pltpu.SemaphoreType.DMA,
            pltpu.SemaphoreType.DMA,
            pltpu.SemaphoreType.DMA,
        ],
    )
    def emb_kernel(idx_hbm, lut_hbm, out_hbm, idx_v, buf0, buf1, buf2, buf3,
                   buf4, buf5, buf6, buf7, gsem0, gsem1, gsem2, gsem3,
                   gsem4, gsem5, gsem6, gsem7, wsem0, wsem1, wsem2, wsem3,
                   wsem4, wsem5, wsem6, wsem7):
        wid = lax.axis_index("s") * nc + lax.axis_index("c")
        base = wid * b_per_w
        pltpu.sync_copy(idx_hbm.at[pl.ds(wid * n_chunks, n_chunks)], idx_v)

        bufs = (buf0, buf1, buf2, buf3, buf4, buf5, buf6, buf7)
        gsems = (gsem0, gsem1, gsem2, gsem3, gsem4, gsem5, gsem6, gsem7)
        wsems = (wsem0, wsem1, wsem2, wsem3, wsem4, wsem5, wsem6, wsem7)

        def gather_start(c, b):
            pltpu.async_copy(lut_hbm.at[idx_v.at[c]], bufs[b], gsems[b])

        def gather_wait(b):
            pltpu.make_async_copy(lut_hbm.at[idx_v.at[0]], bufs[b],
                                  gsems[b]).wait()

        def scale_buf(b):
            buf = bufs[b]

            def rbody(r8, carry):
                for rr in range(8):
                    r = r8 * 8 + rr
                    for j in range(4):
                        sl = pl.ds(16 * j, 16)
                        buf[r, sl] = buf[r, sl] * SCALE
                return carry

            lax.fori_loop(0, CHUNK // 8, rbody, 0)

        def write_start(c, b):
            pltpu.async_copy(bufs[b],
                             out_hbm.at[pl.ds(base + c * CHUNK, CHUNK)],
                             wsems[b])

        def write_wait(b):
            pltpu.make_async_copy(bufs[b], out_hbm.at[pl.ds(0, CHUNK)],
                                  wsems[b]).wait()

        for b in range(NBUF):
            gather_start(b, b)

        def body(t, carry):
            c0 = NBUF * t
            for k in range(NBUF):
                c = c0 + k
                gather_wait(k)
                scale_buf(k)
                write_start(c, k)
                write_wait(k)
                gather_start(c + NBUF, k)
            return carry

        lax.fori_loop(0, n_chunks // NBUF - 1, body, 0)

        c0 = n_chunks - NBUF
        for k in range(NBUF):
            gather_wait(k)
            scale_buf(k)
            write_start(c0 + k, k)
        for k in range(NBUF):
            write_wait(k)

    return emb_kernel


def kernel(x, lut):
    R, C = x.shape
    B = R * C
    xi = x.reshape(-1).astype(jnp.int32)
    idx2 = xi.reshape(B // CHUNK, CHUNK)
    out = _build_gather(B, lut.shape[0])(idx2, lut)
    return out.reshape(R, C, D_MODEL)
